# baseline (speedup 1.0000x reference)
import functools

import jax
import jax.numpy as jnp
from jax.experimental import pallas as pl
from jax.experimental.pallas import tpu as pltpu

_EPS = 1e-6
_COMMITMENT = 1e-4


def _dist_block_kernel(x_ref, means_ref, dists_ref, dmin_ref):
    x = x_ref[0]
    m = means_ref[0]
    n2 = jnp.sum(x * x, axis=1, keepdims=True)
    nrm = jnp.sqrt(n2)
    inv = 1.0 / (nrm + _EPS)
    xn2 = n2 * (inv * inv)
    xa = jnp.concatenate(
        [x * (-2.0 * inv), xn2, jnp.ones_like(xn2)], axis=1)
    m2 = jnp.sum(m * m, axis=1, keepdims=True)
    ma = jnp.concatenate([m, jnp.ones_like(m2), m2], axis=1)
    d2 = jax.lax.dot_general(xa, ma, (((1,), (1,)), ((), ())),
                             preferred_element_type=jnp.float32)
    dists_ref[0] = jnp.sqrt(jnp.maximum(d2, 0.0))
    dmin_ref[0, 0] = jnp.min(d2, axis=1)[None, :]


def _loss_reduce_kernel(dmin_ref, out_ref, *, d):
    n_tokens = dmin_ref.shape[0] * dmin_ref.shape[1]
    scale = _COMMITMENT / float(n_tokens * d)
    s = jnp.sum(dmin_ref[...])
    out_ref[...] = jnp.broadcast_to(s * scale, out_ref.shape)


def kernel(x, means):
    H, T, D = x.shape
    C = means.shape[1]
    Tt = 1024
    nT = T // Tt

    dists, dmin = pl.pallas_call(
        _dist_block_kernel,
        grid=(H, nT),
        in_specs=[
            pl.BlockSpec((1, Tt, D), lambda h, t: (h, t, 0)),
            pl.BlockSpec((1, C, D), lambda h, t: (h, 0, 0)),
        ],
        out_specs=[
            pl.BlockSpec((1, Tt, C), lambda h, t: (h, t, 0)),
            pl.BlockSpec((1, 1, 1, Tt), lambda h, t: (h, t, 0, 0)),
        ],
        out_shape=[
            jax.ShapeDtypeStruct((H, T, C), jnp.float32),
            jax.ShapeDtypeStruct((H, nT, 1, Tt), jnp.float32),
        ],
        compiler_params=pltpu.CompilerParams(
            dimension_semantics=("parallel", "parallel")),
    )(x, means)

    loss_tile = pl.pallas_call(
        functools.partial(_loss_reduce_kernel, d=D),
        out_shape=jax.ShapeDtypeStruct((8, 128), jnp.float32),
    )(dmin.reshape(H * nT, Tt))
    return dists, loss_tile[0, 0]


# speedup vs baseline: 1.4725x; 1.4725x over previous
import functools

import jax
import jax.numpy as jnp
from jax.experimental import pallas as pl
from jax.experimental.pallas import tpu as pltpu

_EPS = 1e-6
_COMMITMENT = 1e-4


def _dist_block_kernel(x_ref, means_ref, dists_ref, dmin_ref):
    x = x_ref[0]
    m = means_ref[0]
    n2 = jnp.sum(x * x, axis=1, keepdims=True)
    inv = 1.0 / (jnp.sqrt(n2) + _EPS)
    x2 = n2 * (inv * inv)
    xs = x * (-2.0 * inv)
    m2 = jnp.sum(m * m, axis=1)[None, :]
    xm = jax.lax.dot_general(xs, m, (((1,), (1,)), ((), ())),
                             preferred_element_type=jnp.float32)
    d2 = (x2 + m2) + xm
    dc = jnp.maximum(d2, 1e-36)
    dists_ref[0] = dc * jax.lax.rsqrt(dc)
    s = jnp.sum(jnp.min(d2, axis=1))
    dmin_ref[0, 0] = jnp.broadcast_to(s, (1, 128))


def _loss_reduce_kernel(dmin_ref, out_ref, *, scale):
    s = jnp.sum(dmin_ref[...])
    out_ref[...] = jnp.broadcast_to(s * scale, out_ref.shape)


def kernel(x, means):
    H, T, D = x.shape
    C = means.shape[1]
    Tt = 1024
    nT = T // Tt

    dists, dmin = pl.pallas_call(
        _dist_block_kernel,
        grid=(H, nT),
        in_specs=[
            pl.BlockSpec((1, Tt, D), lambda h, t: (h, t, 0)),
            pl.BlockSpec((1, C, D), lambda h, t: (h, 0, 0)),
        ],
        out_specs=[
            pl.BlockSpec((1, Tt, C), lambda h, t: (h, t, 0)),
            pl.BlockSpec((1, 1, 1, 128), lambda h, t: (h, t, 0, 0)),
        ],
        out_shape=[
            jax.ShapeDtypeStruct((H, T, C), jnp.float32),
            jax.ShapeDtypeStruct((H, nT, 1, 128), jnp.float32),
        ],
        compiler_params=pltpu.CompilerParams(
            dimension_semantics=("parallel", "parallel")),
    )(x, means)

    loss_tile = pl.pallas_call(
        functools.partial(_loss_reduce_kernel,
                          scale=_COMMITMENT / float(H * T * D * 128)),
        out_shape=jax.ShapeDtypeStruct((8, 128), jnp.float32),
    )(dmin.reshape(H * nT, 128))
    return dists, loss_tile[0, 0]


# speedup vs baseline: 1.5633x; 1.0617x over previous
import functools

import jax
import jax.numpy as jnp
from jax.experimental import pallas as pl
from jax.experimental.pallas import tpu as pltpu

_EPS = 1e-6
_COMMITMENT = 1e-4


def _dist_block_kernel(x_ref, means_ref, dists_ref, dmin_ref):
    x = x_ref[0]
    m = means_ref[0]
    n2 = jnp.sum(x * x, axis=1, keepdims=True)
    nc = jnp.maximum(n2, 1e-36)
    r = jax.lax.rsqrt(nc * jax.lax.rsqrt(nc) + _EPS)
    inv = r * r
    x2 = n2 * (inv * inv)
    xs = x * (-2.0 * inv)
    ones_t = jnp.ones_like(x2)
    xa = jnp.concatenate([xs, ones_t, x2], axis=1)
    m2 = jnp.sum(m * m, axis=1, keepdims=True)
    ones_c = jnp.ones_like(m2)
    ma = jnp.concatenate([m, m2, ones_c], axis=1)
    d2 = jax.lax.dot_general(xa, ma, (((1,), (1,)), ((), ())),
                             preferred_element_type=jnp.float32)
    dc = jnp.maximum(d2, 1e-36)
    dists_ref[0] = dc * jax.lax.rsqrt(dc)
    s = jnp.sum(jnp.min(dc, axis=1))
    sv = jnp.broadcast_to(s, (128,))
    t_idx = pl.program_id(1)

    @pl.when(t_idx == 0)
    def _init():
        dmin_ref[0, 0] = sv

    @pl.when(t_idx != 0)
    def _acc():
        dmin_ref[0, 0] += sv


def _loss_reduce_kernel(dmin_ref, out_ref, *, scale):
    s = jnp.sum(dmin_ref[...])
    out_ref[...] = jnp.broadcast_to(s * scale, out_ref.shape)


def kernel(x, means):
    H, T, D = x.shape
    C = means.shape[1]
    Tt = 1024
    nT = T // Tt

    dists, dmin = pl.pallas_call(
        _dist_block_kernel,
        grid=(H, nT),
        in_specs=[
            pl.BlockSpec((1, Tt, D), lambda h, t: (h, t, 0)),
            pl.BlockSpec((1, C, D), lambda h, t: (h, 0, 0)),
        ],
        out_specs=[
            pl.BlockSpec((1, Tt, C), lambda h, t: (h, t, 0)),
            pl.BlockSpec((1, 1, 128), lambda h, t: (h, 0, 0)),
        ],
        out_shape=[
            jax.ShapeDtypeStruct((H, T, C), jnp.float32),
            jax.ShapeDtypeStruct((H, 1, 128), jnp.float32),
        ],
        compiler_params=pltpu.CompilerParams(
            dimension_semantics=("parallel", "arbitrary")),
    )(x, means)

    loss_tile = pl.pallas_call(
        functools.partial(_loss_reduce_kernel,
                          scale=_COMMITMENT / float(H * T * D * 128)),
        out_shape=jax.ShapeDtypeStruct((8, 128), jnp.float32),
    )(dmin.reshape(H, 128))
    return dists, loss_tile[0, 0]


# speedup vs baseline: 1.9315x; 1.2356x over previous
import functools

import jax
import jax.numpy as jnp
from jax.experimental import pallas as pl
from jax.experimental.pallas import tpu as pltpu

_EPS = 1e-6
_COMMITMENT = 1e-4


def _dist_block_kernel(x_ref, means_ref, dists_ref, dmin_ref):
    x = x_ref[0]
    m = means_ref[0]
    n2 = jnp.sum(x * x, axis=1, keepdims=True)
    nc = jnp.maximum(n2, 1e-36)
    r = jax.lax.rsqrt(nc * jax.lax.rsqrt(nc) + _EPS)
    inv = r * r
    x2 = n2 * (inv * inv)
    xs = x * (-2.0 * inv)
    ones_t = jnp.ones_like(x2)
    xa = jnp.concatenate([xs, ones_t, x2], axis=1)
    m2 = jnp.sum(m * m, axis=1, keepdims=True)
    ones_c = jnp.ones_like(m2)
    ma = jnp.concatenate([m, m2, ones_c], axis=1)
    d2 = jax.lax.dot_general(xa, ma, (((1,), (1,)), ((), ())),
                             preferred_element_type=jnp.float32)
    dc = jnp.maximum(d2, 1e-36)
    dists_ref[0] = dc * jax.lax.rsqrt(dc)
    s = jnp.sum(jnp.min(dc, axis=1))
    sv = jnp.broadcast_to(s, (128,))
    t_idx = pl.program_id(1)

    @pl.when(t_idx == 0)
    def _init():
        dmin_ref[0, 0] = sv

    @pl.when(t_idx != 0)
    def _acc():
        dmin_ref[0, 0] += sv


def _loss_reduce_kernel(dmin_ref, out_ref, *, scale):
    s = jnp.sum(dmin_ref[...])
    out_ref[...] = jnp.broadcast_to(s * scale, out_ref.shape)


def kernel(x, means):
    H, T, D = x.shape
    C = means.shape[1]
    Tt = 2048
    nT = T // Tt

    dists, dmin = pl.pallas_call(
        _dist_block_kernel,
        grid=(H, nT),
        in_specs=[
            pl.BlockSpec((1, Tt, D), lambda h, t: (h, t, 0)),
            pl.BlockSpec((1, C, D), lambda h, t: (h, 0, 0)),
        ],
        out_specs=[
            pl.BlockSpec((1, Tt, C), lambda h, t: (h, t, 0)),
            pl.BlockSpec((1, 1, 128), lambda h, t: (h, 0, 0)),
        ],
        out_shape=[
            jax.ShapeDtypeStruct((H, T, C), jnp.float32),
            jax.ShapeDtypeStruct((H, 1, 128), jnp.float32),
        ],
        compiler_params=pltpu.CompilerParams(
            dimension_semantics=("parallel", "arbitrary")),
    )(x, means)

    loss_tile = pl.pallas_call(
        functools.partial(_loss_reduce_kernel,
                          scale=_COMMITMENT / float(H * T * D * 128)),
        out_shape=jax.ShapeDtypeStruct((8, 128), jnp.float32),
    )(dmin.reshape(H, 128))
    return dists, loss_tile[0, 0]


# speedup vs baseline: 2.2187x; 1.1487x over previous
import functools

import jax
import jax.numpy as jnp
from jax.experimental import pallas as pl
from jax.experimental.pallas import tpu as pltpu

_EPS = 1e-6
_COMMITMENT = 1e-4


def _dist_block_kernel(x_ref, means_ref, dists_ref, dmin_ref):
    x = x_ref[0]
    m = means_ref[0]
    n2 = jnp.sum(x * x, axis=1, keepdims=True)
    nc = jnp.maximum(n2, 1e-36)
    r = jax.lax.rsqrt(nc * jax.lax.rsqrt(nc) + _EPS)
    inv = r * r
    x2 = n2 * (inv * inv)
    xs = x * (-2.0 * inv)
    ones_t = jnp.ones_like(x2)
    xa = jnp.concatenate([xs, ones_t, x2], axis=1)
    m2 = jnp.sum(m * m, axis=1, keepdims=True)
    ones_c = jnp.ones_like(m2)
    ma = jnp.concatenate([m, m2, ones_c], axis=1)
    d2 = jax.lax.dot_general(xa, ma, (((1,), (1,)), ((), ())),
                             preferred_element_type=jnp.float32)
    dc = jnp.maximum(d2, 1e-36)
    dists_ref[0] = dc * jax.lax.rsqrt(dc)
    s = jnp.sum(jnp.min(dc, axis=1))
    sv = jnp.broadcast_to(s, (128,))
    t_idx = pl.program_id(1)

    @pl.when(t_idx == 0)
    def _init():
        dmin_ref[0, 0] = sv

    @pl.when(t_idx != 0)
    def _acc():
        dmin_ref[0, 0] += sv


def _loss_reduce_kernel(dmin_ref, out_ref, *, scale):
    s = jnp.sum(dmin_ref[...])
    out_ref[...] = jnp.broadcast_to(s * scale, out_ref.shape)


def kernel(x, means):
    H, T, D = x.shape
    C = means.shape[1]
    Tt = 4096
    nT = T // Tt

    dists, dmin = pl.pallas_call(
        _dist_block_kernel,
        grid=(H, nT),
        in_specs=[
            pl.BlockSpec((1, Tt, D), lambda h, t: (h, t, 0)),
            pl.BlockSpec((1, C, D), lambda h, t: (h, 0, 0)),
        ],
        out_specs=[
            pl.BlockSpec((1, Tt, C), lambda h, t: (h, t, 0)),
            pl.BlockSpec((1, 1, 128), lambda h, t: (h, 0, 0)),
        ],
        out_shape=[
            jax.ShapeDtypeStruct((H, T, C), jnp.float32),
            jax.ShapeDtypeStruct((H, 1, 128), jnp.float32),
        ],
        compiler_params=pltpu.CompilerParams(
            dimension_semantics=("parallel", "arbitrary")),
    )(x, means)

    loss_tile = pl.pallas_call(
        functools.partial(_loss_reduce_kernel,
                          scale=_COMMITMENT / float(H * T * D * 128)),
        out_shape=jax.ShapeDtypeStruct((8, 128), jnp.float32),
    )(dmin.reshape(H, 128))
    return dists, loss_tile[0, 0]


# speedup vs baseline: 2.3055x; 1.0391x over previous
import functools

import jax
import jax.numpy as jnp
from jax.experimental import pallas as pl
from jax.experimental.pallas import tpu as pltpu

_EPS = 1e-6
_COMMITMENT = 1e-4


def _dist_block_kernel(x_ref, means_ref, dists_ref, dmin_ref):
    x = x_ref[0]
    m = means_ref[0]
    n2 = jnp.sum(x * x, axis=1, keepdims=True)
    nc = jnp.maximum(n2, 1e-36)
    r = jax.lax.rsqrt(nc * jax.lax.rsqrt(nc) + _EPS)
    inv = r * r
    x2 = n2 * (inv * inv)
    xs = x * (-2.0 * inv)
    ones_t = jnp.ones_like(x2)
    xa = jnp.concatenate([xs, ones_t, x2], axis=1)
    m2 = jnp.sum(m * m, axis=1, keepdims=True)
    ones_c = jnp.ones_like(m2)
    ma = jnp.concatenate([m, m2, ones_c], axis=1)
    d2 = jax.lax.dot_general(xa, ma, (((1,), (1,)), ((), ())),
                             preferred_element_type=jnp.float32)
    dc = jnp.maximum(d2, 1e-36)
    dists_ref[0] = dc * jax.lax.rsqrt(dc)
    s = jnp.sum(jnp.min(dc, axis=1))
    sv = jnp.broadcast_to(s, (128,))
    t_idx = pl.program_id(1)

    @pl.when(t_idx == 0)
    def _init():
        dmin_ref[0, 0] = sv

    @pl.when(t_idx != 0)
    def _acc():
        dmin_ref[0, 0] += sv


def _loss_reduce_kernel(dmin_ref, out_ref, *, scale):
    s = jnp.sum(dmin_ref[...])
    out_ref[...] = jnp.broadcast_to(s * scale, out_ref.shape)


def kernel(x, means):
    H, T, D = x.shape
    C = means.shape[1]
    Tt = 8192
    nT = T // Tt

    dists, dmin = pl.pallas_call(
        _dist_block_kernel,
        grid=(H, nT),
        in_specs=[
            pl.BlockSpec((1, Tt, D), lambda h, t: (h, t, 0)),
            pl.BlockSpec((1, C, D), lambda h, t: (h, 0, 0)),
        ],
        out_specs=[
            pl.BlockSpec((1, Tt, C), lambda h, t: (h, t, 0)),
            pl.BlockSpec((1, 1, 128), lambda h, t: (h, 0, 0)),
        ],
        out_shape=[
            jax.ShapeDtypeStruct((H, T, C), jnp.float32),
            jax.ShapeDtypeStruct((H, 1, 128), jnp.float32),
        ],
        compiler_params=pltpu.CompilerParams(
            dimension_semantics=("parallel", "arbitrary")),
    )(x, means)

    loss_tile = pl.pallas_call(
        functools.partial(_loss_reduce_kernel,
                          scale=_COMMITMENT / float(H * T * D * 128)),
        out_shape=jax.ShapeDtypeStruct((8, 128), jnp.float32),
    )(dmin.reshape(H, 128))
    return dists, loss_tile[0, 0]
